# baseline (speedup 1.0000x reference)
import functools

import jax
import jax.numpy as jnp
from jax import lax
from jax.experimental import pallas as pl
from jax.experimental.pallas import tpu as pltpu
from jax.experimental.pallas import tpu_sc as plsc

_N_E = 8192
_E_DIM = 256
_B = 16384
_BETA = 0.25
_BM = 256
_NB = _B // _BM
_EPS = 1e-12


def _norm_body(e_ref, o_ref):
    e = e_ref[...]
    n = jnp.sqrt(jnp.sum(e * e, axis=1, keepdims=True))
    o_ref[...] = e / jnp.maximum(n, _EPS)


def _norm_call(emb):
    return pl.pallas_call(
        _norm_body,
        grid=(8,),
        in_specs=[pl.BlockSpec((_N_E // 8, _E_DIM), lambda i: (i, 0))],
        out_specs=pl.BlockSpec((_N_E // 8, _E_DIM), lambda i: (i, 0)),
        out_shape=jax.ShapeDtypeStruct((_N_E, _E_DIM), jnp.float32),
        compiler_params=pltpu.CompilerParams(
            dimension_semantics=("parallel",)),
    )(emb)


def _argmin_body(x_ref, cbn_ref, idx_ref):
    x = x_ref[...]
    xn = x / jnp.maximum(jnp.sqrt(jnp.sum(x * x, axis=1, keepdims=True)), _EPS)
    sim = lax.dot_general(xn, cbn_ref[...], (((1,), (1,)), ((), ())),
                          preferred_element_type=jnp.float32)
    _C = 128
    cur = 1.0 - sim[:, 0:_C]
    cur_j = jnp.zeros((_BM, _C), jnp.int32)
    for j in range(1, _N_E // _C):
        dj = 1.0 - sim[:, j * _C:(j + 1) * _C]
        lt = dj < cur
        cur = jnp.where(lt, dj, cur)
        cur_j = jnp.where(lt, j, cur_j)
    lane = lax.broadcasted_iota(jnp.int32, (_BM, _C), 1)
    col = cur_j * _C + lane
    m = jnp.min(cur, axis=1, keepdims=True)
    idx = jnp.min(jnp.where(cur == m, col, _N_E), axis=1)
    idx_ref[0, 0, :] = idx.astype(jnp.int32)


def _argmin_call(x, cbn):
    return pl.pallas_call(
        _argmin_body,
        grid=(_NB,),
        in_specs=[
            pl.BlockSpec((_BM, _E_DIM), lambda i: (i, 0)),
            pl.BlockSpec((_N_E, _E_DIM), lambda i: (0, 0)),
        ],
        out_specs=pl.BlockSpec((1, 1, _BM), lambda i: (i, 0, 0)),
        out_shape=jax.ShapeDtypeStruct((_NB, 1, _BM), jnp.int32),
        compiler_params=pltpu.CompilerParams(
            dimension_semantics=("parallel",)),
    )(x, cbn)


_NCORES = 2
_NSUB = 16
_NW = _NCORES * _NSUB
_CH = 128
_NCH = _B // (_NW * _CH)


@functools.cache
def _make_gather_sc():
    @functools.partial(
        pl.kernel,
        mesh=plsc.VectorSubcoreMesh(core_axis_name="c", subcore_axis_name="s"),
        out_type=jax.ShapeDtypeStruct((_B, _E_DIM), jnp.float32),
        scratch_types=[
            pltpu.VMEM((_CH,), jnp.int32),
            pltpu.VMEM((_CH, _E_DIM), jnp.float32),
            pltpu.SemaphoreType.DMA,
        ],
    )
    def _gather_sc(emb_hbm, idx_hbm, out_hbm, idx_v, rows_v, sem):
        wid = lax.axis_index("s") * _NCORES + lax.axis_index("c")
        for j in range(_NCH):
            pltpu.sync_copy(idx_hbm.at[wid, j], idx_v)
            pltpu.async_copy(emb_hbm.at[idx_v], rows_v, sem).wait()
            pltpu.sync_copy(rows_v,
                            out_hbm.at[pl.ds((wid * _NCH + j) * _CH, _CH)])

    return _gather_sc


def _proj_body(x_ref, cv_ref, xq_ref, sc_ref, loss_ref, acc_ref):
    @pl.when(pl.program_id(0) == 0)
    def _():
        acc_ref[0, 0] = 0.0

    x = x_ref[...]
    cv = cv_ref[...]
    dot = jnp.sum(x * cv, axis=1, keepdims=True)
    nsq = jnp.sum(cv * cv, axis=1, keepdims=True)
    scalar = dot / (nsq + 1e-08)
    proj = scalar * cv
    xq_ref[...] = x + (proj - x)
    sc_ref[0, 0, :] = scalar[:, 0]
    acc_ref[0, 0] += jnp.sum((proj - x) ** 2)

    @pl.when(pl.program_id(0) == _NB - 1)
    def _():
        m = acc_ref[0, 0] / (_B * _E_DIM)
        loss_ref[...] = jnp.reshape(m + _BETA * m, (1, 1))


def _proj_call(x, cv):
    return pl.pallas_call(
        _proj_body,
        grid=(_NB,),
        in_specs=[
            pl.BlockSpec((_BM, _E_DIM), lambda i: (i, 0)),
            pl.BlockSpec((_BM, _E_DIM), lambda i: (i, 0)),
        ],
        out_specs=[
            pl.BlockSpec((_BM, _E_DIM), lambda i: (i, 0)),
            pl.BlockSpec((1, 1, _BM), lambda i: (i, 0, 0)),
            pl.BlockSpec((1, 1), lambda i: (0, 0)),
        ],
        out_shape=[
            jax.ShapeDtypeStruct((_B, _E_DIM), jnp.float32),
            jax.ShapeDtypeStruct((_NB, 1, _BM), jnp.float32),
            jax.ShapeDtypeStruct((1, 1), jnp.float32),
        ],
        scratch_shapes=[pltpu.SMEM((1, 1), jnp.float32)],
    )(x, cv)


def kernel(x, emb):
    idx3 = _argmin_call(x, _norm_call(emb))
    indices = idx3.reshape(_B)
    cv = _make_gather_sc()(emb, indices.reshape(_NW, _NCH, _CH))
    xq, sc3, loss11 = _proj_call(x, cv)
    return (xq, loss11[0, 0], indices, sc3.reshape(_B))


# speedup vs baseline: 1.2106x; 1.2106x over previous
import functools

import jax
import jax.numpy as jnp
from jax import lax
from jax.experimental import pallas as pl
from jax.experimental.pallas import tpu as pltpu
from jax.experimental.pallas import tpu_sc as plsc

_N_E = 8192
_E_DIM = 256
_B = 16384
_BETA = 0.25
_BM = 256
_NB = _B // _BM
_EPS = 1e-12


def _argmin_body(x_ref, emb_ref, idx_ref, cbn_ref):
    @pl.when(pl.program_id(0) == 0)
    def _():
        e = emb_ref[...]
        n = jnp.sqrt(jnp.sum(e * e, axis=1, keepdims=True))
        cbn_ref[...] = e / jnp.maximum(n, _EPS)

    x = x_ref[...]
    xn = x / jnp.maximum(jnp.sqrt(jnp.sum(x * x, axis=1, keepdims=True)), _EPS)
    _C = 128
    _CW = 2048
    cur = None
    for c in range(_N_E // _CW):
        sim = lax.dot_general(xn, cbn_ref[c * _CW:(c + 1) * _CW, :],
                              (((1,), (1,)), ((), ())),
                              preferred_element_type=jnp.float32)
        for j in range(_CW // _C):
            sj = sim[:, j * _C:(j + 1) * _C]
            jj = c * (_CW // _C) + j
            if cur is None:
                cur = sj
                cur_j = jnp.zeros((_BM, _C), jnp.int32)
            else:
                gt = sj > cur
                cur = jnp.where(gt, sj, cur)
                cur_j = jnp.where(gt, jj, cur_j)
    lane = lax.broadcasted_iota(jnp.int32, (_BM, _C), 1)
    col = cur_j * _C + lane
    m = jnp.max(cur, axis=1, keepdims=True)
    idx = jnp.min(jnp.where(cur == m, col, _N_E), axis=1)
    idx_ref[0, 0, :] = idx.astype(jnp.int32)


def _argmin_call(x, emb):
    return pl.pallas_call(
        _argmin_body,
        grid=(_NB,),
        in_specs=[
            pl.BlockSpec((_BM, _E_DIM), lambda i: (i, 0)),
            pl.BlockSpec((_N_E, _E_DIM), lambda i: (0, 0)),
        ],
        out_specs=pl.BlockSpec((1, 1, _BM), lambda i: (i, 0, 0)),
        out_shape=jax.ShapeDtypeStruct((_NB, 1, _BM), jnp.int32),
        scratch_shapes=[pltpu.VMEM((_N_E, _E_DIM), jnp.float32)],
    )(x, emb)


_NCORES = 2
_NSUB = 16
_NW = _NCORES * _NSUB
_CH = 128
_NCH = _B // (_NW * _CH)


@functools.cache
def _make_gather_sc():
    @functools.partial(
        pl.kernel,
        mesh=plsc.VectorSubcoreMesh(core_axis_name="c", subcore_axis_name="s"),
        out_type=jax.ShapeDtypeStruct((_B, _E_DIM), jnp.float32),
        scratch_types=[
            pltpu.VMEM((_CH,), jnp.int32),
            pltpu.VMEM((_CH, _E_DIM), jnp.float32),
            pltpu.SemaphoreType.DMA,
        ],
    )
    def _gather_sc(emb_hbm, idx_hbm, out_hbm, idx_v, rows_v, sem):
        wid = lax.axis_index("s") * _NCORES + lax.axis_index("c")
        for j in range(_NCH):
            pltpu.sync_copy(idx_hbm.at[wid, j], idx_v)
            pltpu.async_copy(emb_hbm.at[idx_v], rows_v, sem).wait()
            pltpu.sync_copy(rows_v,
                            out_hbm.at[pl.ds((wid * _NCH + j) * _CH, _CH)])

    return _gather_sc


def _proj_body(x_ref, cv_ref, xq_ref, sc_ref, loss_ref, acc_ref):
    @pl.when(pl.program_id(0) == 0)
    def _():
        acc_ref[0, 0] = 0.0

    x = x_ref[...]
    cv = cv_ref[...]
    dot = jnp.sum(x * cv, axis=1, keepdims=True)
    nsq = jnp.sum(cv * cv, axis=1, keepdims=True)
    scalar = dot / (nsq + 1e-08)
    proj = scalar * cv
    xq_ref[...] = x + (proj - x)
    sc_ref[0, 0, :] = scalar[:, 0]
    acc_ref[0, 0] += jnp.sum((proj - x) ** 2)

    @pl.when(pl.program_id(0) == _NB - 1)
    def _():
        m = acc_ref[0, 0] / (_B * _E_DIM)
        loss_ref[...] = jnp.reshape(m + _BETA * m, (1, 1))


def _proj_call(x, cv):
    return pl.pallas_call(
        _proj_body,
        grid=(_NB,),
        in_specs=[
            pl.BlockSpec((_BM, _E_DIM), lambda i: (i, 0)),
            pl.BlockSpec((_BM, _E_DIM), lambda i: (i, 0)),
        ],
        out_specs=[
            pl.BlockSpec((_BM, _E_DIM), lambda i: (i, 0)),
            pl.BlockSpec((1, 1, _BM), lambda i: (i, 0, 0)),
            pl.BlockSpec((1, 1), lambda i: (0, 0)),
        ],
        out_shape=[
            jax.ShapeDtypeStruct((_B, _E_DIM), jnp.float32),
            jax.ShapeDtypeStruct((_NB, 1, _BM), jnp.float32),
            jax.ShapeDtypeStruct((1, 1), jnp.float32),
        ],
        scratch_shapes=[pltpu.SMEM((1, 1), jnp.float32)],
    )(x, cv)


def kernel(x, emb):
    idx3 = _argmin_call(x, emb)
    indices = idx3.reshape(_B)
    cv = _make_gather_sc()(emb, indices.reshape(_NW, _NCH, _CH))
    xq, sc3, loss11 = _proj_call(x, cv)
    return (xq, loss11[0, 0], indices, sc3.reshape(_B))


# speedup vs baseline: 1.2173x; 1.0055x over previous
import functools

import jax
import jax.numpy as jnp
from jax import lax
from jax.experimental import pallas as pl
from jax.experimental.pallas import tpu as pltpu
from jax.experimental.pallas import tpu_sc as plsc

_N_E = 8192
_E_DIM = 256
_B = 16384
_BETA = 0.25
_BM = 256
_NB = _B // _BM
_EPS = 1e-12


def _argmin_body(x_ref, emb_ref, idx_ref, cbn_ref):
    @pl.when(pl.program_id(0) == 0)
    def _():
        e = emb_ref[...]
        n = jnp.sqrt(jnp.sum(e * e, axis=1, keepdims=True))
        cbn_ref[...] = e / jnp.maximum(n, _EPS)

    x = x_ref[...]
    xn = x / jnp.maximum(jnp.sqrt(jnp.sum(x * x, axis=1, keepdims=True)), _EPS)
    _C = 128
    _CW = 2048
    cur = None
    for c in range(_N_E // _CW):
        sim = lax.dot_general(xn, cbn_ref[c * _CW:(c + 1) * _CW, :],
                              (((1,), (1,)), ((), ())),
                              preferred_element_type=jnp.float32)
        for j in range(_CW // _C):
            sj = sim[:, j * _C:(j + 1) * _C]
            jj = c * (_CW // _C) + j
            if cur is None:
                cur = sj
                cur_j = jnp.zeros((_BM, _C), jnp.int32)
            else:
                gt = sj > cur
                cur = jnp.where(gt, sj, cur)
                cur_j = jnp.where(gt, jj, cur_j)
    lane = lax.broadcasted_iota(jnp.int32, (_BM, _C), 1)
    col = cur_j * _C + lane
    m = jnp.max(cur, axis=1, keepdims=True)
    idx = jnp.min(jnp.where(cur == m, col, _N_E), axis=1)
    idx_ref[0, 0, :] = idx.astype(jnp.int32)


def _argmin_call(x, emb):
    return pl.pallas_call(
        _argmin_body,
        grid=(_NB,),
        in_specs=[
            pl.BlockSpec((_BM, _E_DIM), lambda i: (i, 0)),
            pl.BlockSpec((_N_E, _E_DIM), lambda i: (0, 0)),
        ],
        out_specs=pl.BlockSpec((1, 1, _BM), lambda i: (i, 0, 0)),
        out_shape=jax.ShapeDtypeStruct((_NB, 1, _BM), jnp.int32),
        scratch_shapes=[pltpu.VMEM((_N_E, _E_DIM), jnp.float32)],
    )(x, emb)


_NCORES = 2
_NSUB = 16
_NW = _NCORES * _NSUB
_CH = 128
_NCH = _B // (_NW * _CH)


@functools.cache
def _make_gather_sc():
    @functools.partial(
        pl.kernel,
        mesh=plsc.VectorSubcoreMesh(core_axis_name="c", subcore_axis_name="s"),
        out_type=jax.ShapeDtypeStruct((_B, _E_DIM), jnp.float32),
        scratch_types=[
            pltpu.VMEM((_NCH, _CH), jnp.int32),
            pltpu.VMEM((_CH, _E_DIM), jnp.float32),
            pltpu.VMEM((_CH, _E_DIM), jnp.float32),
            pltpu.SemaphoreType.DMA,
            pltpu.SemaphoreType.DMA,
        ],
    )
    def _gather_sc(emb_hbm, idx_hbm, out_hbm, idx_v, rows0, rows1, s0, s1):
        wid = lax.axis_index("s") * _NCORES + lax.axis_index("c")
        base = wid * _NCH * _CH
        rows = (rows0, rows1)
        sems = (s0, s1)
        pltpu.sync_copy(idx_hbm.at[wid], idx_v)
        cps = [None, None]
        cps[0] = pltpu.async_copy(emb_hbm.at[idx_v.at[0]], rows0, s0)
        cps[1] = pltpu.async_copy(emb_hbm.at[idx_v.at[1]], rows1, s1)
        for j in range(_NCH):
            cps[j % 2].wait()
            pltpu.sync_copy(rows[j % 2], out_hbm.at[pl.ds(base + j * _CH, _CH)])
            if j + 2 < _NCH:
                cps[j % 2] = pltpu.async_copy(
                    emb_hbm.at[idx_v.at[j + 2]], rows[j % 2], sems[j % 2])

    return _gather_sc


def _proj_body(x_ref, cv_ref, xq_ref, sc_ref, loss_ref, acc_ref):
    @pl.when(pl.program_id(0) == 0)
    def _():
        acc_ref[0, 0] = 0.0

    x = x_ref[...]
    cv = cv_ref[...]
    dot = jnp.sum(x * cv, axis=1, keepdims=True)
    nsq = jnp.sum(cv * cv, axis=1, keepdims=True)
    scalar = dot / (nsq + 1e-08)
    proj = scalar * cv
    xq_ref[...] = x + (proj - x)
    sc_ref[0, 0, :] = scalar[:, 0]
    acc_ref[0, 0] += jnp.sum((proj - x) ** 2)

    @pl.when(pl.program_id(0) == _NB - 1)
    def _():
        m = acc_ref[0, 0] / (_B * _E_DIM)
        loss_ref[...] = jnp.reshape(m + _BETA * m, (1, 1))


def _proj_call(x, cv):
    return pl.pallas_call(
        _proj_body,
        grid=(_NB,),
        in_specs=[
            pl.BlockSpec((_BM, _E_DIM), lambda i: (i, 0)),
            pl.BlockSpec((_BM, _E_DIM), lambda i: (i, 0)),
        ],
        out_specs=[
            pl.BlockSpec((_BM, _E_DIM), lambda i: (i, 0)),
            pl.BlockSpec((1, 1, _BM), lambda i: (i, 0, 0)),
            pl.BlockSpec((1, 1), lambda i: (0, 0)),
        ],
        out_shape=[
            jax.ShapeDtypeStruct((_B, _E_DIM), jnp.float32),
            jax.ShapeDtypeStruct((_NB, 1, _BM), jnp.float32),
            jax.ShapeDtypeStruct((1, 1), jnp.float32),
        ],
        scratch_shapes=[pltpu.SMEM((1, 1), jnp.float32)],
    )(x, cv)


def kernel(x, emb):
    idx3 = _argmin_call(x, emb)
    indices = idx3.reshape(_B)
    cv = _make_gather_sc()(emb, indices.reshape(_NW, _NCH, _CH))
    xq, sc3, loss11 = _proj_call(x, cv)
    return (xq, loss11[0, 0], indices, sc3.reshape(_B))


# speedup vs baseline: 1.3789x; 1.1327x over previous
import functools

import jax
import jax.numpy as jnp
from jax import lax
from jax.experimental import pallas as pl
from jax.experimental.pallas import tpu as pltpu
from jax.experimental.pallas import tpu_sc as plsc

_N_E = 8192
_E_DIM = 256
_B = 16384
_BETA = 0.25
_BM = 512
_NB = _B // _BM
_EPS = 1e-12


def _argmin_body(x_ref, emb_ref, idx_ref, cbn_ref):
    @pl.when(pl.program_id(0) == 0)
    def _():
        e = emb_ref[...]
        n = jnp.sqrt(jnp.sum(e * e, axis=1, keepdims=True))
        cbn_ref[...] = e / jnp.maximum(n, _EPS)

    x = x_ref[...]
    xn = x / jnp.maximum(jnp.sqrt(jnp.sum(x * x, axis=1, keepdims=True)), _EPS)
    _C = 128
    _CW = 2048
    cur = None
    for c in range(_N_E // _CW):
        sim = lax.dot_general(xn, cbn_ref[c * _CW:(c + 1) * _CW, :],
                              (((1,), (1,)), ((), ())),
                              preferred_element_type=jnp.float32)
        for j in range(_CW // _C):
            sj = sim[:, j * _C:(j + 1) * _C]
            jj = c * (_CW // _C) + j
            if cur is None:
                cur = sj
                cur_j = jnp.zeros((_BM, _C), jnp.int32)
            else:
                gt = sj > cur
                cur = jnp.where(gt, sj, cur)
                cur_j = jnp.where(gt, jj, cur_j)
    lane = lax.broadcasted_iota(jnp.int32, (_BM, _C), 1)
    col = cur_j * _C + lane
    m = jnp.max(cur, axis=1, keepdims=True)
    idx = jnp.min(jnp.where(cur == m, col, _N_E), axis=1)
    idx_ref[0, 0, :] = idx.astype(jnp.int32)


def _argmin_call(x, emb):
    return pl.pallas_call(
        _argmin_body,
        grid=(_NB,),
        in_specs=[
            pl.BlockSpec((_BM, _E_DIM), lambda i: (i, 0)),
            pl.BlockSpec((_N_E, _E_DIM), lambda i: (0, 0)),
        ],
        out_specs=pl.BlockSpec((1, 1, _BM), lambda i: (i, 0, 0)),
        out_shape=jax.ShapeDtypeStruct((_NB, 1, _BM), jnp.int32),
        scratch_shapes=[pltpu.VMEM((_N_E, _E_DIM), jnp.float32)],
    )(x, emb)


_NCORES = 2
_NSUB = 16
_NW = _NCORES * _NSUB
_CH = 128
_NCH = _B // (_NW * _CH)


@functools.cache
def _make_gather_sc():
    @functools.partial(
        pl.kernel,
        mesh=plsc.VectorSubcoreMesh(core_axis_name="c", subcore_axis_name="s"),
        out_type=jax.ShapeDtypeStruct((_B, _E_DIM), jnp.float32),
        scratch_types=[
            pltpu.VMEM((_NCH, _CH), jnp.int32),
            pltpu.VMEM((_CH, _E_DIM), jnp.float32),
            pltpu.VMEM((_CH, _E_DIM), jnp.float32),
            pltpu.SemaphoreType.DMA,
            pltpu.SemaphoreType.DMA,
        ],
    )
    def _gather_sc(emb_hbm, idx_hbm, out_hbm, idx_v, rows0, rows1, s0, s1):
        wid = lax.axis_index("s") * _NCORES + lax.axis_index("c")
        base = wid * _NCH * _CH
        rows = (rows0, rows1)
        sems = (s0, s1)
        pltpu.sync_copy(idx_hbm.at[wid], idx_v)
        cps = [None, None]
        cps[0] = pltpu.async_copy(emb_hbm.at[idx_v.at[0]], rows0, s0)
        cps[1] = pltpu.async_copy(emb_hbm.at[idx_v.at[1]], rows1, s1)
        for j in range(_NCH):
            cps[j % 2].wait()
            pltpu.sync_copy(rows[j % 2], out_hbm.at[pl.ds(base + j * _CH, _CH)])
            if j + 2 < _NCH:
                cps[j % 2] = pltpu.async_copy(
                    emb_hbm.at[idx_v.at[j + 2]], rows[j % 2], sems[j % 2])

    return _gather_sc


def _proj_body(x_ref, cv_ref, xq_ref, sc_ref, loss_ref, acc_ref):
    @pl.when(pl.program_id(0) == 0)
    def _():
        acc_ref[0, 0] = 0.0

    x = x_ref[...]
    cv = cv_ref[...]
    dot = jnp.sum(x * cv, axis=1, keepdims=True)
    nsq = jnp.sum(cv * cv, axis=1, keepdims=True)
    scalar = dot / (nsq + 1e-08)
    proj = scalar * cv
    xq_ref[...] = x + (proj - x)
    sc_ref[0, 0, :] = scalar[:, 0]
    acc_ref[0, 0] += jnp.sum((proj - x) ** 2)

    @pl.when(pl.program_id(0) == _NB - 1)
    def _():
        m = acc_ref[0, 0] / (_B * _E_DIM)
        loss_ref[...] = jnp.reshape(m + _BETA * m, (1, 1))


def _proj_call(x, cv):
    return pl.pallas_call(
        _proj_body,
        grid=(_NB,),
        in_specs=[
            pl.BlockSpec((_BM, _E_DIM), lambda i: (i, 0)),
            pl.BlockSpec((_BM, _E_DIM), lambda i: (i, 0)),
        ],
        out_specs=[
            pl.BlockSpec((_BM, _E_DIM), lambda i: (i, 0)),
            pl.BlockSpec((1, 1, _BM), lambda i: (i, 0, 0)),
            pl.BlockSpec((1, 1), lambda i: (0, 0)),
        ],
        out_shape=[
            jax.ShapeDtypeStruct((_B, _E_DIM), jnp.float32),
            jax.ShapeDtypeStruct((_NB, 1, _BM), jnp.float32),
            jax.ShapeDtypeStruct((1, 1), jnp.float32),
        ],
        scratch_shapes=[pltpu.SMEM((1, 1), jnp.float32)],
    )(x, cv)


def kernel(x, emb):
    idx3 = _argmin_call(x, emb)
    indices = idx3.reshape(_B)
    cv = _make_gather_sc()(emb, indices.reshape(_NW, _NCH, _CH))
    xq, sc3, loss11 = _proj_call(x, cv)
    return (xq, loss11[0, 0], indices, sc3.reshape(_B))


# speedup vs baseline: 1.4744x; 1.0693x over previous
import functools

import jax
import jax.numpy as jnp
from jax import lax
from jax.experimental import pallas as pl
from jax.experimental.pallas import tpu as pltpu
from jax.experimental.pallas import tpu_sc as plsc

_N_E = 8192
_E_DIM = 256
_B = 16384
_BETA = 0.25
_BM = 1024
_NB = _B // _BM
_EPS = 1e-12


def _argmin_body(x_ref, emb_ref, idx_ref, cbn_ref):
    @pl.when(pl.program_id(0) == 0)
    def _():
        e = emb_ref[...]
        n = jnp.sqrt(jnp.sum(e * e, axis=1, keepdims=True))
        cbn_ref[...] = e / jnp.maximum(n, _EPS)

    x = x_ref[...]
    xn = x / jnp.maximum(jnp.sqrt(jnp.sum(x * x, axis=1, keepdims=True)), _EPS)
    _C = 128
    _CW = 2048
    cur = None
    for c in range(_N_E // _CW):
        sim = lax.dot_general(xn, cbn_ref[c * _CW:(c + 1) * _CW, :],
                              (((1,), (1,)), ((), ())),
                              preferred_element_type=jnp.float32)
        for j in range(_CW // _C):
            sj = sim[:, j * _C:(j + 1) * _C]
            jj = c * (_CW // _C) + j
            if cur is None:
                cur = sj
                cur_j = jnp.zeros((_BM, _C), jnp.int32)
            else:
                gt = sj > cur
                cur = jnp.where(gt, sj, cur)
                cur_j = jnp.where(gt, jj, cur_j)
    lane = lax.broadcasted_iota(jnp.int32, (_BM, _C), 1)
    col = cur_j * _C + lane
    m = jnp.max(cur, axis=1, keepdims=True)
    idx = jnp.min(jnp.where(cur == m, col, _N_E), axis=1)
    idx_ref[0, 0, :] = idx.astype(jnp.int32)


def _argmin_call(x, emb):
    return pl.pallas_call(
        _argmin_body,
        grid=(_NB,),
        in_specs=[
            pl.BlockSpec((_BM, _E_DIM), lambda i: (i, 0)),
            pl.BlockSpec((_N_E, _E_DIM), lambda i: (0, 0)),
        ],
        out_specs=pl.BlockSpec((1, 1, _BM), lambda i: (i, 0, 0)),
        out_shape=jax.ShapeDtypeStruct((_NB, 1, _BM), jnp.int32),
        scratch_shapes=[pltpu.VMEM((_N_E, _E_DIM), jnp.float32)],
    )(x, emb)


_NCORES = 2
_NSUB = 16
_NW = _NCORES * _NSUB
_CH = 128
_NCH = _B // (_NW * _CH)


@functools.cache
def _make_gather_sc():
    @functools.partial(
        pl.kernel,
        mesh=plsc.VectorSubcoreMesh(core_axis_name="c", subcore_axis_name="s"),
        out_type=jax.ShapeDtypeStruct((_B, _E_DIM), jnp.float32),
        scratch_types=[
            pltpu.VMEM((_NCH, _CH), jnp.int32),
            pltpu.VMEM((_CH, _E_DIM), jnp.float32),
            pltpu.VMEM((_CH, _E_DIM), jnp.float32),
            pltpu.SemaphoreType.DMA,
            pltpu.SemaphoreType.DMA,
        ],
    )
    def _gather_sc(emb_hbm, idx_hbm, out_hbm, idx_v, rows0, rows1, s0, s1):
        wid = lax.axis_index("s") * _NCORES + lax.axis_index("c")
        base = wid * _NCH * _CH
        rows = (rows0, rows1)
        sems = (s0, s1)
        pltpu.sync_copy(idx_hbm.at[wid], idx_v)
        cps = [None, None]
        cps[0] = pltpu.async_copy(emb_hbm.at[idx_v.at[0]], rows0, s0)
        cps[1] = pltpu.async_copy(emb_hbm.at[idx_v.at[1]], rows1, s1)
        for j in range(_NCH):
            cps[j % 2].wait()
            pltpu.sync_copy(rows[j % 2], out_hbm.at[pl.ds(base + j * _CH, _CH)])
            if j + 2 < _NCH:
                cps[j % 2] = pltpu.async_copy(
                    emb_hbm.at[idx_v.at[j + 2]], rows[j % 2], sems[j % 2])

    return _gather_sc


def _proj_body(x_ref, cv_ref, xq_ref, sc_ref, loss_ref, acc_ref):
    @pl.when(pl.program_id(0) == 0)
    def _():
        acc_ref[0, 0] = 0.0

    x = x_ref[...]
    cv = cv_ref[...]
    dot = jnp.sum(x * cv, axis=1, keepdims=True)
    nsq = jnp.sum(cv * cv, axis=1, keepdims=True)
    scalar = dot / (nsq + 1e-08)
    proj = scalar * cv
    xq_ref[...] = x + (proj - x)
    sc_ref[0, 0, :] = scalar[:, 0]
    acc_ref[0, 0] += jnp.sum((proj - x) ** 2)

    @pl.when(pl.program_id(0) == _NB - 1)
    def _():
        m = acc_ref[0, 0] / (_B * _E_DIM)
        loss_ref[...] = jnp.reshape(m + _BETA * m, (1, 1))


def _proj_call(x, cv):
    return pl.pallas_call(
        _proj_body,
        grid=(_NB,),
        in_specs=[
            pl.BlockSpec((_BM, _E_DIM), lambda i: (i, 0)),
            pl.BlockSpec((_BM, _E_DIM), lambda i: (i, 0)),
        ],
        out_specs=[
            pl.BlockSpec((_BM, _E_DIM), lambda i: (i, 0)),
            pl.BlockSpec((1, 1, _BM), lambda i: (i, 0, 0)),
            pl.BlockSpec((1, 1), lambda i: (0, 0)),
        ],
        out_shape=[
            jax.ShapeDtypeStruct((_B, _E_DIM), jnp.float32),
            jax.ShapeDtypeStruct((_NB, 1, _BM), jnp.float32),
            jax.ShapeDtypeStruct((1, 1), jnp.float32),
        ],
        scratch_shapes=[pltpu.SMEM((1, 1), jnp.float32)],
    )(x, cv)


def kernel(x, emb):
    idx3 = _argmin_call(x, emb)
    indices = idx3.reshape(_B)
    cv = _make_gather_sc()(emb, indices.reshape(_NW, _NCH, _CH))
    xq, sc3, loss11 = _proj_call(x, cv)
    return (xq, loss11[0, 0], indices, sc3.reshape(_B))


# speedup vs baseline: 1.5157x; 1.0280x over previous
import functools

import jax
import jax.numpy as jnp
from jax import lax
from jax.experimental import pallas as pl
from jax.experimental.pallas import tpu as pltpu
from jax.experimental.pallas import tpu_sc as plsc

_N_E = 8192
_E_DIM = 256
_B = 16384
_BETA = 0.25
_BM = 2048
_NB = _B // _BM
_EPS = 1e-12


def _argmin_body(x_ref, emb_ref, idx_ref, cbn_ref):
    @pl.when(pl.program_id(0) == 0)
    def _():
        e = emb_ref[...]
        n = jnp.sqrt(jnp.sum(e * e, axis=1, keepdims=True))
        cbn_ref[...] = e / jnp.maximum(n, _EPS)

    x = x_ref[...]
    xn = x / jnp.maximum(jnp.sqrt(jnp.sum(x * x, axis=1, keepdims=True)), _EPS)
    _C = 128
    _CW = 2048
    cur = None
    for c in range(_N_E // _CW):
        sim = lax.dot_general(xn, cbn_ref[c * _CW:(c + 1) * _CW, :],
                              (((1,), (1,)), ((), ())),
                              preferred_element_type=jnp.float32)
        for j in range(_CW // _C):
            sj = sim[:, j * _C:(j + 1) * _C]
            jj = c * (_CW // _C) + j
            if cur is None:
                cur = sj
                cur_j = jnp.zeros((_BM, _C), jnp.int32)
            else:
                gt = sj > cur
                cur = jnp.where(gt, sj, cur)
                cur_j = jnp.where(gt, jj, cur_j)
    lane = lax.broadcasted_iota(jnp.int32, (_BM, _C), 1)
    col = cur_j * _C + lane
    m = jnp.max(cur, axis=1, keepdims=True)
    idx = jnp.min(jnp.where(cur == m, col, _N_E), axis=1)
    idx_ref[0, 0, :] = idx.astype(jnp.int32)


def _argmin_call(x, emb):
    return pl.pallas_call(
        _argmin_body,
        grid=(_NB,),
        in_specs=[
            pl.BlockSpec((_BM, _E_DIM), lambda i: (i, 0)),
            pl.BlockSpec((_N_E, _E_DIM), lambda i: (0, 0)),
        ],
        out_specs=pl.BlockSpec((1, 1, _BM), lambda i: (i, 0, 0)),
        out_shape=jax.ShapeDtypeStruct((_NB, 1, _BM), jnp.int32),
        scratch_shapes=[pltpu.VMEM((_N_E, _E_DIM), jnp.float32)],
    )(x, emb)


_NCORES = 2
_NSUB = 16
_NW = _NCORES * _NSUB
_CH = 128
_NCH = _B // (_NW * _CH)


@functools.cache
def _make_gather_sc():
    @functools.partial(
        pl.kernel,
        mesh=plsc.VectorSubcoreMesh(core_axis_name="c", subcore_axis_name="s"),
        out_type=jax.ShapeDtypeStruct((_B, _E_DIM), jnp.float32),
        scratch_types=[
            pltpu.VMEM((_NCH, _CH), jnp.int32),
            pltpu.VMEM((_CH, _E_DIM), jnp.float32),
            pltpu.VMEM((_CH, _E_DIM), jnp.float32),
            pltpu.SemaphoreType.DMA,
            pltpu.SemaphoreType.DMA,
        ],
    )
    def _gather_sc(emb_hbm, idx_hbm, out_hbm, idx_v, rows0, rows1, s0, s1):
        wid = lax.axis_index("s") * _NCORES + lax.axis_index("c")
        base = wid * _NCH * _CH
        rows = (rows0, rows1)
        sems = (s0, s1)
        pltpu.sync_copy(idx_hbm.at[wid], idx_v)
        cps = [None, None]
        cps[0] = pltpu.async_copy(emb_hbm.at[idx_v.at[0]], rows0, s0)
        cps[1] = pltpu.async_copy(emb_hbm.at[idx_v.at[1]], rows1, s1)
        for j in range(_NCH):
            cps[j % 2].wait()
            pltpu.sync_copy(rows[j % 2], out_hbm.at[pl.ds(base + j * _CH, _CH)])
            if j + 2 < _NCH:
                cps[j % 2] = pltpu.async_copy(
                    emb_hbm.at[idx_v.at[j + 2]], rows[j % 2], sems[j % 2])

    return _gather_sc


def _proj_body(x_ref, cv_ref, xq_ref, sc_ref, loss_ref, acc_ref):
    @pl.when(pl.program_id(0) == 0)
    def _():
        acc_ref[0, 0] = 0.0

    x = x_ref[...]
    cv = cv_ref[...]
    dot = jnp.sum(x * cv, axis=1, keepdims=True)
    nsq = jnp.sum(cv * cv, axis=1, keepdims=True)
    scalar = dot / (nsq + 1e-08)
    proj = scalar * cv
    xq_ref[...] = x + (proj - x)
    sc_ref[0, 0, :] = scalar[:, 0]
    acc_ref[0, 0] += jnp.sum((proj - x) ** 2)

    @pl.when(pl.program_id(0) == _NB - 1)
    def _():
        m = acc_ref[0, 0] / (_B * _E_DIM)
        loss_ref[...] = jnp.reshape(m + _BETA * m, (1, 1))


def _proj_call(x, cv):
    return pl.pallas_call(
        _proj_body,
        grid=(_NB,),
        in_specs=[
            pl.BlockSpec((_BM, _E_DIM), lambda i: (i, 0)),
            pl.BlockSpec((_BM, _E_DIM), lambda i: (i, 0)),
        ],
        out_specs=[
            pl.BlockSpec((_BM, _E_DIM), lambda i: (i, 0)),
            pl.BlockSpec((1, 1, _BM), lambda i: (i, 0, 0)),
            pl.BlockSpec((1, 1), lambda i: (0, 0)),
        ],
        out_shape=[
            jax.ShapeDtypeStruct((_B, _E_DIM), jnp.float32),
            jax.ShapeDtypeStruct((_NB, 1, _BM), jnp.float32),
            jax.ShapeDtypeStruct((1, 1), jnp.float32),
        ],
        scratch_shapes=[pltpu.SMEM((1, 1), jnp.float32)],
    )(x, cv)


def kernel(x, emb):
    idx3 = _argmin_call(x, emb)
    indices = idx3.reshape(_B)
    cv = _make_gather_sc()(emb, indices.reshape(_NW, _NCH, _CH))
    xq, sc3, loss11 = _proj_call(x, cv)
    return (xq, loss11[0, 0], indices, sc3.reshape(_B))


# speedup vs baseline: 1.5166x; 1.0006x over previous
import functools

import jax
import jax.numpy as jnp
from jax import lax
from jax.experimental import pallas as pl
from jax.experimental.pallas import tpu as pltpu
from jax.experimental.pallas import tpu_sc as plsc

_N_E = 8192
_E_DIM = 256
_B = 16384
_BETA = 0.25
_BM = 2048
_NB = _B // _BM
_EPS = 1e-12


def _argmin_body(x_ref, emb_ref, idx_ref, cbn_ref):
    @pl.when(pl.program_id(0) == 0)
    def _():
        e = emb_ref[...]
        n = jnp.sqrt(jnp.sum(e * e, axis=1, keepdims=True))
        cbn_ref[...] = e / jnp.maximum(n, _EPS)

    x = x_ref[...]
    xn = x / jnp.maximum(jnp.sqrt(jnp.sum(x * x, axis=1, keepdims=True)), _EPS)
    _C = 128
    _CW = 1024
    cur = None
    for c in range(_N_E // _CW):
        sim = lax.dot_general(xn, cbn_ref[c * _CW:(c + 1) * _CW, :],
                              (((1,), (1,)), ((), ())),
                              preferred_element_type=jnp.float32)
        for j in range(_CW // _C):
            sj = sim[:, j * _C:(j + 1) * _C]
            jj = c * (_CW // _C) + j
            if cur is None:
                cur = sj
                cur_j = jnp.zeros((_BM, _C), jnp.int32)
            else:
                gt = sj > cur
                cur = jnp.where(gt, sj, cur)
                cur_j = jnp.where(gt, jj, cur_j)
    lane = lax.broadcasted_iota(jnp.int32, (_BM, _C), 1)
    col = cur_j * _C + lane
    m = jnp.max(cur, axis=1, keepdims=True)
    idx = jnp.min(jnp.where(cur == m, col, _N_E), axis=1)
    idx_ref[0, 0, :] = idx.astype(jnp.int32)


def _argmin_call(x, emb):
    return pl.pallas_call(
        _argmin_body,
        grid=(_NB,),
        in_specs=[
            pl.BlockSpec((_BM, _E_DIM), lambda i: (i, 0)),
            pl.BlockSpec((_N_E, _E_DIM), lambda i: (0, 0)),
        ],
        out_specs=pl.BlockSpec((1, 1, _BM), lambda i: (i, 0, 0)),
        out_shape=jax.ShapeDtypeStruct((_NB, 1, _BM), jnp.int32),
        scratch_shapes=[pltpu.VMEM((_N_E, _E_DIM), jnp.float32)],
    )(x, emb)


_NCORES = 2
_NSUB = 16
_NW = _NCORES * _NSUB
_CH = 128
_NCH = _B // (_NW * _CH)


@functools.cache
def _make_gather_sc():
    @functools.partial(
        pl.kernel,
        mesh=plsc.VectorSubcoreMesh(core_axis_name="c", subcore_axis_name="s"),
        out_type=jax.ShapeDtypeStruct((_B, _E_DIM), jnp.float32),
        scratch_types=[
            pltpu.VMEM((_NCH, _CH), jnp.int32),
            pltpu.VMEM((_CH, _E_DIM), jnp.float32),
            pltpu.VMEM((_CH, _E_DIM), jnp.float32),
            pltpu.SemaphoreType.DMA,
            pltpu.SemaphoreType.DMA,
        ],
    )
    def _gather_sc(emb_hbm, idx_hbm, out_hbm, idx_v, rows0, rows1, s0, s1):
        wid = lax.axis_index("s") * _NCORES + lax.axis_index("c")
        base = wid * _NCH * _CH
        rows = (rows0, rows1)
        sems = (s0, s1)
        pltpu.sync_copy(idx_hbm.at[wid], idx_v)
        cps = [None, None]
        cps[0] = pltpu.async_copy(emb_hbm.at[idx_v.at[0]], rows0, s0)
        cps[1] = pltpu.async_copy(emb_hbm.at[idx_v.at[1]], rows1, s1)
        for j in range(_NCH):
            cps[j % 2].wait()
            pltpu.sync_copy(rows[j % 2], out_hbm.at[pl.ds(base + j * _CH, _CH)])
            if j + 2 < _NCH:
                cps[j % 2] = pltpu.async_copy(
                    emb_hbm.at[idx_v.at[j + 2]], rows[j % 2], sems[j % 2])

    return _gather_sc


def _proj_body(x_ref, cv_ref, xq_ref, sc_ref, loss_ref, acc_ref):
    @pl.when(pl.program_id(0) == 0)
    def _():
        acc_ref[0, 0] = 0.0

    x = x_ref[...]
    cv = cv_ref[...]
    dot = jnp.sum(x * cv, axis=1, keepdims=True)
    nsq = jnp.sum(cv * cv, axis=1, keepdims=True)
    scalar = dot / (nsq + 1e-08)
    proj = scalar * cv
    xq_ref[...] = x + (proj - x)
    sc_ref[0, 0, :] = scalar[:, 0]
    acc_ref[0, 0] += jnp.sum((proj - x) ** 2)

    @pl.when(pl.program_id(0) == _NB - 1)
    def _():
        m = acc_ref[0, 0] / (_B * _E_DIM)
        loss_ref[...] = jnp.reshape(m + _BETA * m, (1, 1))


def _proj_call(x, cv):
    return pl.pallas_call(
        _proj_body,
        grid=(_NB,),
        in_specs=[
            pl.BlockSpec((_BM, _E_DIM), lambda i: (i, 0)),
            pl.BlockSpec((_BM, _E_DIM), lambda i: (i, 0)),
        ],
        out_specs=[
            pl.BlockSpec((_BM, _E_DIM), lambda i: (i, 0)),
            pl.BlockSpec((1, 1, _BM), lambda i: (i, 0, 0)),
            pl.BlockSpec((1, 1), lambda i: (0, 0)),
        ],
        out_shape=[
            jax.ShapeDtypeStruct((_B, _E_DIM), jnp.float32),
            jax.ShapeDtypeStruct((_NB, 1, _BM), jnp.float32),
            jax.ShapeDtypeStruct((1, 1), jnp.float32),
        ],
        scratch_shapes=[pltpu.SMEM((1, 1), jnp.float32)],
    )(x, cv)


def kernel(x, emb):
    idx3 = _argmin_call(x, emb)
    indices = idx3.reshape(_B)
    cv = _make_gather_sc()(emb, indices.reshape(_NW, _NCH, _CH))
    xq, sc3, loss11 = _proj_call(x, cv)
    return (xq, loss11[0, 0], indices, sc3.reshape(_B))


# speedup vs baseline: 1.6429x; 1.0833x over previous
import functools

import jax
import jax.numpy as jnp
from jax import lax
from jax.experimental import pallas as pl
from jax.experimental.pallas import tpu as pltpu
from jax.experimental.pallas import tpu_sc as plsc

_N_E = 8192
_E_DIM = 256
_B = 16384
_BETA = 0.25
_BM = 2048
_NB = _B // _BM
_EPS = 1e-12


def _argmin_body(x_ref, emb_ref, idx_ref, cbn_ref):
    @pl.when(pl.program_id(0) == 0)
    def _():
        e = emb_ref[...]
        n = jnp.sqrt(jnp.sum(e * e, axis=1, keepdims=True))
        cbn_ref[...] = e / jnp.maximum(n, _EPS)

    x = x_ref[...]
    xn = x / jnp.maximum(jnp.sqrt(jnp.sum(x * x, axis=1, keepdims=True)), _EPS)
    _C = 128
    _CW = 1024
    _RS = 512
    lane = lax.broadcasted_iota(jnp.int32, (_RS, _C), 1)
    for r in range(_BM // _RS):
        xr = xn[r * _RS:(r + 1) * _RS, :]
        cur = None
        for c in range(_N_E // _CW):
            sim = lax.dot_general(xr, cbn_ref[c * _CW:(c + 1) * _CW, :],
                                  (((1,), (1,)), ((), ())),
                                  preferred_element_type=jnp.float32)
            for j in range(_CW // _C):
                sj = sim[:, j * _C:(j + 1) * _C]
                jj = c * (_CW // _C) + j
                if cur is None:
                    cur = sj
                    cur_j = jnp.zeros((_RS, _C), jnp.int32)
                else:
                    gt = sj > cur
                    cur = jnp.where(gt, sj, cur)
                    cur_j = jnp.where(gt, jj, cur_j)
        col = cur_j * _C + lane
        m = jnp.max(cur, axis=1, keepdims=True)
        idx = jnp.min(jnp.where(cur == m, col, _N_E), axis=1)
        idx_ref[0, 0, r * _RS:(r + 1) * _RS] = idx.astype(jnp.int32)


def _argmin_call(x, emb):
    return pl.pallas_call(
        _argmin_body,
        grid=(_NB,),
        in_specs=[
            pl.BlockSpec((_BM, _E_DIM), lambda i: (i, 0)),
            pl.BlockSpec((_N_E, _E_DIM), lambda i: (0, 0)),
        ],
        out_specs=pl.BlockSpec((1, 1, _BM), lambda i: (i, 0, 0)),
        out_shape=jax.ShapeDtypeStruct((_NB, 1, _BM), jnp.int32),
        scratch_shapes=[pltpu.VMEM((_N_E, _E_DIM), jnp.float32)],
    )(x, emb)


_NCORES = 2
_NSUB = 16
_NW = _NCORES * _NSUB
_CH = 128
_NCH = _B // (_NW * _CH)


@functools.cache
def _make_gather_sc():
    @functools.partial(
        pl.kernel,
        mesh=plsc.VectorSubcoreMesh(core_axis_name="c", subcore_axis_name="s"),
        out_type=jax.ShapeDtypeStruct((_B, _E_DIM), jnp.float32),
        scratch_types=[
            pltpu.VMEM((_NCH, _CH), jnp.int32),
            pltpu.VMEM((_CH, _E_DIM), jnp.float32),
            pltpu.VMEM((_CH, _E_DIM), jnp.float32),
            pltpu.SemaphoreType.DMA,
            pltpu.SemaphoreType.DMA,
        ],
    )
    def _gather_sc(emb_hbm, idx_hbm, out_hbm, idx_v, rows0, rows1, s0, s1):
        wid = lax.axis_index("s") * _NCORES + lax.axis_index("c")
        base = wid * _NCH * _CH
        rows = (rows0, rows1)
        sems = (s0, s1)
        pltpu.sync_copy(idx_hbm.at[wid], idx_v)
        cps = [None, None]
        cps[0] = pltpu.async_copy(emb_hbm.at[idx_v.at[0]], rows0, s0)
        cps[1] = pltpu.async_copy(emb_hbm.at[idx_v.at[1]], rows1, s1)
        for j in range(_NCH):
            cps[j % 2].wait()
            pltpu.sync_copy(rows[j % 2], out_hbm.at[pl.ds(base + j * _CH, _CH)])
            if j + 2 < _NCH:
                cps[j % 2] = pltpu.async_copy(
                    emb_hbm.at[idx_v.at[j + 2]], rows[j % 2], sems[j % 2])

    return _gather_sc


def _proj_body(x_ref, cv_ref, xq_ref, sc_ref, loss_ref, acc_ref):
    @pl.when(pl.program_id(0) == 0)
    def _():
        acc_ref[0, 0] = 0.0

    x = x_ref[...]
    cv = cv_ref[...]
    dot = jnp.sum(x * cv, axis=1, keepdims=True)
    nsq = jnp.sum(cv * cv, axis=1, keepdims=True)
    scalar = dot / (nsq + 1e-08)
    proj = scalar * cv
    xq_ref[...] = x + (proj - x)
    sc_ref[0, 0, :] = scalar[:, 0]
    acc_ref[0, 0] += jnp.sum((proj - x) ** 2)

    @pl.when(pl.program_id(0) == _NB - 1)
    def _():
        m = acc_ref[0, 0] / (_B * _E_DIM)
        loss_ref[...] = jnp.reshape(m + _BETA * m, (1, 1))


def _proj_call(x, cv):
    return pl.pallas_call(
        _proj_body,
        grid=(_NB,),
        in_specs=[
            pl.BlockSpec((_BM, _E_DIM), lambda i: (i, 0)),
            pl.BlockSpec((_BM, _E_DIM), lambda i: (i, 0)),
        ],
        out_specs=[
            pl.BlockSpec((_BM, _E_DIM), lambda i: (i, 0)),
            pl.BlockSpec((1, 1, _BM), lambda i: (i, 0, 0)),
            pl.BlockSpec((1, 1), lambda i: (0, 0)),
        ],
        out_shape=[
            jax.ShapeDtypeStruct((_B, _E_DIM), jnp.float32),
            jax.ShapeDtypeStruct((_NB, 1, _BM), jnp.float32),
            jax.ShapeDtypeStruct((1, 1), jnp.float32),
        ],
        scratch_shapes=[pltpu.SMEM((1, 1), jnp.float32)],
    )(x, cv)


def kernel(x, emb):
    idx3 = _argmin_call(x, emb)
    indices = idx3.reshape(_B)
    cv = _make_gather_sc()(emb, indices.reshape(_NW, _NCH, _CH))
    xq, sc3, loss11 = _proj_call(x, cv)
    return (xq, loss11[0, 0], indices, sc3.reshape(_B))
